# Initial kernel scaffold; baseline (speedup 1.0000x reference)
import jax
import jax.numpy as jnp
from jax.experimental import pallas as pl


def kernel(input):
    raise NotImplementedError("write your pallas kernel here")


# speedup vs baseline: 1.7364x; 1.7364x over previous
import jax
import jax.numpy as jnp
from jax.experimental import pallas as pl

_ROWS = 16384
_BLOCK_ROWS = 1024


def _copy_kernel(x_ref, o_ref):
    o_ref[...] = x_ref[...]


def kernel(input):
    n = _ROWS // _BLOCK_ROWS
    x = input.reshape(_ROWS, 32 * 128)
    out = pl.pallas_call(
        _copy_kernel,
        grid=(n,),
        in_specs=[
            pl.BlockSpec((_BLOCK_ROWS, 3 * 128), lambda i: (i, 5)),
        ],
        out_specs=pl.BlockSpec((_BLOCK_ROWS, 3 * 128), lambda i: (i, 0)),
        out_shape=jax.ShapeDtypeStruct((_ROWS, 3 * 128), input.dtype),
    )(x)
    return out.reshape(_ROWS, 3, 128)


# speedup vs baseline: 1.7664x; 1.0173x over previous
import jax
import jax.numpy as jnp
from jax.experimental import pallas as pl

_ROWS = 16384
_BLOCK_ROWS = 4096


def _copy_kernel(x_ref, o_ref):
    o_ref[...] = x_ref[...]


def kernel(input):
    n = _ROWS // _BLOCK_ROWS
    x = input.reshape(_ROWS, 32 * 128)
    out = pl.pallas_call(
        _copy_kernel,
        grid=(n,),
        in_specs=[
            pl.BlockSpec((_BLOCK_ROWS, 3 * 128), lambda i: (i, 5)),
        ],
        out_specs=pl.BlockSpec((_BLOCK_ROWS, 3 * 128), lambda i: (i, 0)),
        out_shape=jax.ShapeDtypeStruct((_ROWS, 3 * 128), input.dtype),
    )(x)
    return out.reshape(_ROWS, 3, 128)
